# Initial kernel scaffold; baseline (speedup 1.0000x reference)
import jax
import jax.numpy as jnp
from jax.experimental import pallas as pl


def kernel(x, edge_index, batch, x_loc, params):
    raise NotImplementedError("write your pallas kernel here")


# speedup vs baseline: 16.2733x; 16.2733x over previous
import functools

import jax
import jax.numpy as jnp
import numpy as np
from jax import lax
from jax.experimental import pallas as pl
from jax.experimental.pallas import tpu as pltpu
from jax.experimental.pallas import tpu_sc as plsc

_N = 10000
_NPAD = 10240
_E = 320000
_DIN = 128
_H = 64
_G = 64
_NC = 2
_NS = 16
_NW = _NC * _NS
_EPW = _E // _NW
_CHUNK = 128
_CH = (_EPW + _CHUNK - 1) // _CHUNK
_EPW_PAD = _CH * _CHUNK
_RPT = _NPAD // _NS
_BLK = 256
_NBLK = _NPAD // _BLK
_BN_R = float(1.0 / np.sqrt(1.0 + 1e-5))


def _sc_mesh():
    return plsc.VectorSubcoreMesh(core_axis_name="c", subcore_axis_name="s")


def _deg_body(dst_hbm, out0, out1, dstv, buf, acc):
    c = lax.axis_index("c")
    sid = lax.axis_index("s")
    w = c * _NS + sid
    base = sid * _RPT

    def _zrow(i, carry):
        buf[i, pl.ds(0, 16)] = jnp.zeros((16,), jnp.float32)
        return carry

    lax.fori_loop(0, _CHUNK, _zrow, 0)
    for k in range(_RPT // _CHUNK):
        pltpu.sync_copy(buf, acc.at[pl.ds(base + k * _CHUNK, _CHUNK)])

    def _orow(i, carry):
        buf[i, pl.ds(0, 16)] = jnp.ones((16,), jnp.float32)
        return carry

    lax.fori_loop(0, _CHUNK, _orow, 0)
    pltpu.sync_copy(dst_hbm.at[w], dstv)
    plsc.subcore_barrier()

    def _step(j, carry):
        pltpu.sync_copy(buf, acc.at[dstv.at[j]], add=True)
        return carry

    lax.fori_loop(0, _CH, _step, 0)
    plsc.subcore_barrier()

    @pl.when(c == 0)
    def _():
        pltpu.sync_copy(acc.at[pl.ds(base, _RPT)], out0.at[pl.ds(base, _RPT)])

    @pl.when(c == 1)
    def _():
        pltpu.sync_copy(acc.at[pl.ds(base, _RPT)], out1.at[pl.ds(base, _RPT)])


_deg_call = pl.kernel(
    _deg_body,
    out_type=[jax.ShapeDtypeStruct((_NPAD, 16), jnp.float32)] * 2,
    mesh=_sc_mesh(),
    scratch_types=[
        pltpu.VMEM((_CH, _CHUNK), jnp.int32),
        pltpu.VMEM((_CHUNK, 16), jnp.float32),
        pltpu.VMEM_SHARED((_NPAD, 16), jnp.float32),
    ],
    compiler_params=pltpu.CompilerParams(use_tc_tiling_on_sc=False),
)


def _edge_body(src_hbm, dst_hbm, s_hbm, out0, out1, srcv, dstv, rows, acc, sem):
    c = lax.axis_index("c")
    sid = lax.axis_index("s")
    w = c * _NS + sid
    base = sid * _RPT

    def _zrow(i, carry):
        for k in range(_H // 16):
            rows[i, pl.ds(k * 16, 16)] = jnp.zeros((16,), jnp.float32)
        return carry

    lax.fori_loop(0, _CHUNK, _zrow, 0)
    for k in range(_RPT // _CHUNK):
        pltpu.sync_copy(rows, acc.at[pl.ds(base + k * _CHUNK, _CHUNK)])
    pltpu.sync_copy(src_hbm.at[w], srcv)
    pltpu.sync_copy(dst_hbm.at[w], dstv)
    plsc.subcore_barrier()

    def _step(j, carry):
        pltpu.async_copy(s_hbm.at[srcv.at[j]], rows, sem).wait()
        pltpu.sync_copy(rows, acc.at[dstv.at[j]], add=True)
        return carry

    lax.fori_loop(0, _CH, _step, 0)
    plsc.subcore_barrier()

    @pl.when(c == 0)
    def _():
        pltpu.sync_copy(acc.at[pl.ds(base, _RPT)], out0.at[pl.ds(base, _RPT)])

    @pl.when(c == 1)
    def _():
        pltpu.sync_copy(acc.at[pl.ds(base, _RPT)], out1.at[pl.ds(base, _RPT)])


_edge_call = pl.kernel(
    _edge_body,
    out_type=[jax.ShapeDtypeStruct((_NPAD, _H), jnp.float32)] * 2,
    mesh=_sc_mesh(),
    scratch_types=[
        pltpu.VMEM((_CH, _CHUNK), jnp.int32),
        pltpu.VMEM((_CH, _CHUNK), jnp.int32),
        pltpu.VMEM((_CHUNK, _H), jnp.float32),
        pltpu.VMEM_SHARED((_NPAD, _H), jnp.float32),
        pltpu.SemaphoreType.DMA,
    ],
    compiler_params=pltpu.CompilerParams(use_tc_tiling_on_sc=False),
)


def _prologue_body(x_ref, h0_ref, h1_ref, w_ref, o_ref):
    deg = 1.0 + h0_ref[:, 0:1] + h1_ref[:, 0:1]
    dinv = lax.rsqrt(deg)
    o_ref[...] = (
        jnp.dot(x_ref[...], w_ref[...], preferred_element_type=jnp.float32) * dinv
    )


_prologue_call = pl.pallas_call(
    _prologue_body,
    grid=(_NBLK,),
    in_specs=[
        pl.BlockSpec((_BLK, _DIN), lambda i: (i, 0)),
        pl.BlockSpec((_BLK, 16), lambda i: (i, 0)),
        pl.BlockSpec((_BLK, 16), lambda i: (i, 0)),
        pl.BlockSpec((_DIN, _H), lambda i: (0, 0)),
    ],
    out_specs=pl.BlockSpec((_BLK, _H), lambda i: (i, 0)),
    out_shape=jax.ShapeDtypeStruct((_NPAD, _H), jnp.float32),
)


def _mid_body(a0, a1, s, h0, h1, ga, cb, w_ref, o_ref):
    deg = 1.0 + h0[:, 0:1] + h1[:, 0:1]
    dinv = lax.rsqrt(deg)
    h = jnp.maximum(dinv * (a0[...] + a1[...] + s[...]) * ga[...] + cb[...], 0.0)
    o_ref[...] = (
        jnp.dot(h, w_ref[...], preferred_element_type=jnp.float32) * dinv
    )


_mid_call = pl.pallas_call(
    _mid_body,
    grid=(_NBLK,),
    in_specs=[
        pl.BlockSpec((_BLK, _H), lambda i: (i, 0)),
        pl.BlockSpec((_BLK, _H), lambda i: (i, 0)),
        pl.BlockSpec((_BLK, _H), lambda i: (i, 0)),
        pl.BlockSpec((_BLK, 16), lambda i: (i, 0)),
        pl.BlockSpec((_BLK, 16), lambda i: (i, 0)),
        pl.BlockSpec((1, _H), lambda i: (0, 0)),
        pl.BlockSpec((1, _H), lambda i: (0, 0)),
        pl.BlockSpec((_H, _H), lambda i: (0, 0)),
    ],
    out_specs=pl.BlockSpec((_BLK, _H), lambda i: (i, 0)),
    out_shape=jax.ShapeDtypeStruct((_NPAD, _H), jnp.float32),
)


def _pool_body(a0, a1, s, h0, h1, ga, cb, b_ref, sums, cnts):
    i = pl.program_id(0)
    deg = 1.0 + h0[:, 0:1] + h1[:, 0:1]
    dinv = lax.rsqrt(deg)
    h3 = jnp.maximum(dinv * (a0[...] + a1[...] + s[...]) * ga[...] + cb[...], 0.0)
    gid = lax.broadcasted_iota(jnp.int32, (_BLK, _G), 1)
    oh = (b_ref[...] == gid).astype(jnp.float32)
    dn = (((0,), (0,)), ((), ()))
    ps = lax.dot_general(oh, h3, dn, preferred_element_type=jnp.float32)
    pc = lax.dot_general(
        oh, jnp.ones((_BLK, _G), jnp.float32), dn, preferred_element_type=jnp.float32
    )

    @pl.when(i == 0)
    def _():
        sums[...] = jnp.zeros_like(sums)
        cnts[...] = jnp.zeros_like(cnts)

    sums[...] += ps
    cnts[...] += pc


_pool_call = pl.pallas_call(
    _pool_body,
    grid=(_NBLK,),
    in_specs=[
        pl.BlockSpec((_BLK, _H), lambda i: (i, 0)),
        pl.BlockSpec((_BLK, _H), lambda i: (i, 0)),
        pl.BlockSpec((_BLK, _H), lambda i: (i, 0)),
        pl.BlockSpec((_BLK, 16), lambda i: (i, 0)),
        pl.BlockSpec((_BLK, 16), lambda i: (i, 0)),
        pl.BlockSpec((1, _H), lambda i: (0, 0)),
        pl.BlockSpec((1, _H), lambda i: (0, 0)),
        pl.BlockSpec((_BLK, 1), lambda i: (i, 0)),
    ],
    out_specs=[
        pl.BlockSpec((_G, _H), lambda i: (0, 0)),
        pl.BlockSpec((_G, _G), lambda i: (0, 0)),
    ],
    out_shape=[
        jax.ShapeDtypeStruct((_G, _H), jnp.float32),
        jax.ShapeDtypeStruct((_G, _G), jnp.float32),
    ],
)


def _head_body(sums, cnts, m1w, m1b, m2w, m2b, m3w, m3b, ow, ob,
               t1w, t1b, t2w, t2b, t3w, t3b, xl, bias, o_ref):
    f32 = jnp.float32
    pooled = sums[...] / jnp.maximum(cnts[...], 1.0)
    z = jnp.maximum(jnp.dot(pooled, m1w[...], preferred_element_type=f32) + m1b[...], 0.0)
    z = jnp.maximum(jnp.dot(z, m2w[...], preferred_element_type=f32) + m2b[...], 0.0)
    z = jnp.maximum(jnp.dot(z, m3w[...], preferred_element_type=f32) + m3b[...], 0.0)
    bf = jnp.dot(z, ow[...], preferred_element_type=f32) + ob[...]
    t = jnp.maximum(jnp.dot(xl[...], t1w[...], preferred_element_type=f32) + t1b[...], 0.0)
    t = jnp.maximum(jnp.dot(t, t2w[...], preferred_element_type=f32) + t2b[...], 0.0)
    tf = jnp.dot(t, t3w[...], preferred_element_type=f32) + t3b[...]
    o_ref[...] = bf * tf + bias[...]


_head_call = pl.pallas_call(
    _head_body,
    out_shape=jax.ShapeDtypeStruct((_G, 2), jnp.float32),
)


def kernel(x, edge_index, batch, x_loc, params):
    p = params
    src = edge_index[0].reshape(_NW, _EPW)
    dst = edge_index[1].reshape(_NW, _EPW)
    pad = _EPW_PAD - _EPW
    srcp = jnp.pad(src, ((0, 0), (0, pad))).reshape(_NW, _CH, _CHUNK)
    dstp = jnp.pad(dst, ((0, 0), (0, pad)), constant_values=_N).reshape(
        _NW, _CH, _CHUNK
    )
    xp = jnp.pad(x, ((0, _NPAD - _N), (0, 0)))
    bp = jnp.pad(batch, (0, _NPAD - _N), constant_values=_G).reshape(_NPAD, 1)

    def fold(g, be, b):
        ga = (g * _BN_R).reshape(1, _H)
        cb = (b * g * _BN_R + be).reshape(1, _H)
        return ga, cb

    ga1, cb1 = fold(p["g1"], p["be1"], p["b1"])
    ga2, cb2 = fold(p["g2"], p["be2"], p["b2"])
    ga3, cb3 = fold(p["g3"], p["be3"], p["b3"])

    h0, h1 = _deg_call(dstp)
    s1 = _prologue_call(xp, h0, h1, p["W1"])
    a10, a11 = _edge_call(srcp, dstp, s1)
    s2 = _mid_call(a10, a11, s1, h0, h1, ga1, cb1, p["W2"])
    a20, a21 = _edge_call(srcp, dstp, s2)
    s3 = _mid_call(a20, a21, s2, h0, h1, ga2, cb2, p["W3"])
    a30, a31 = _edge_call(srcp, dstp, s3)
    sums, cnts = _pool_call(a30, a31, s3, h0, h1, ga3, cb3, bp)

    out = _head_call(
        sums, cnts,
        p["m1W"], p["m1b"].reshape(1, -1),
        p["m2W"], p["m2b"].reshape(1, -1),
        p["m3W"], p["m3b"].reshape(1, -1),
        p["oW"], p["ob"].reshape(1, -1),
        p["t1W"], p["t1b"].reshape(1, -1),
        p["t2W"], p["t2b"].reshape(1, -1),
        p["t3W"], p["t3b"].reshape(1, -1),
        x_loc, p["bias"].reshape(1, -1),
    )
    return out
